# baseline (speedup 1.0000x reference)
import functools

import jax
import jax.numpy as jnp
from jax import lax
from jax.experimental import pallas as pl
from jax.experimental.pallas import tpu as pltpu
from jax.experimental.pallas import tpu_sc as plsc

_CHUNK = 128


def kernel(heads, rels, entity_table, rel_table):
    B = heads.shape[0]
    D = entity_table.shape[1]

    info = plsc.get_sparse_core_info()
    NC, NS = info.num_cores, info.num_subcores
    NW = NC * NS
    b_per_w = B // NW
    n_chunks = b_per_w // _CHUNK
    assert b_per_w * NW == B and n_chunks * _CHUNK == b_per_w

    mesh = plsc.VectorSubcoreMesh(core_axis_name="c", subcore_axis_name="s")

    @functools.partial(
        pl.kernel,
        mesh=mesh,
        compiler_params=pltpu.CompilerParams(use_tc_tiling_on_sc=False),
        out_type=(
            jax.ShapeDtypeStruct((B, D), jnp.float32),
            jax.ShapeDtypeStruct((B, D), jnp.float32),
        ),
        scratch_types=[
            pltpu.VMEM((n_chunks, _CHUNK), jnp.int32),
            pltpu.VMEM((b_per_w, D), jnp.float32),
            pltpu.VMEM((n_chunks, _CHUNK), jnp.int32),
            pltpu.VMEM((b_per_w, D), jnp.float32),
            pltpu.SemaphoreType.DMA,
        ],
    )
    def _gather2(heads_hbm, rels_hbm, etab_hbm, rtab_hbm, out_h_hbm, out_r_hbm,
                 hidx_v, hrows_v, ridx_v, rrows_v, sem):
        wid = lax.axis_index("s") * NC + lax.axis_index("c")
        base = wid * b_per_w
        for j in range(n_chunks):
            pltpu.sync_copy(heads_hbm.at[pl.ds(base + j * _CHUNK, _CHUNK)],
                            hidx_v.at[j])
            pltpu.sync_copy(rels_hbm.at[pl.ds(base + j * _CHUNK, _CHUNK)],
                            ridx_v.at[j])
        copies = []
        for j in range(n_chunks):
            copies.append(pltpu.async_copy(
                etab_hbm.at[hidx_v.at[j]],
                hrows_v.at[pl.ds(j * _CHUNK, _CHUNK)], sem))
            copies.append(pltpu.async_copy(
                rtab_hbm.at[ridx_v.at[j]],
                rrows_v.at[pl.ds(j * _CHUNK, _CHUNK)], sem))
        for c in copies:
            c.wait()
        pltpu.sync_copy(hrows_v, out_h_hbm.at[pl.ds(base, b_per_w)])
        pltpu.sync_copy(rrows_v, out_r_hbm.at[pl.ds(base, b_per_w)])

    return _gather2(heads, rels, entity_table, rel_table)
